# baseline (speedup 1.0000x reference)
import functools

import jax
import jax.numpy as jnp
from jax.experimental import pallas as pl
from jax.experimental.pallas import tpu as pltpu

_EPAD = 128


def _body(E, sp_ref, x_ref, w1_ref, b1_ref, w2p_ref, b2p_ref, tab_ref,
          out_ref, acc_ref, log_ref):
    j = pl.program_id(0)
    k = pl.program_id(1)
    nj = pl.num_programs(0)
    nk = pl.num_programs(1)

    part = jnp.dot(x_ref[...], w1_ref[...], preferred_element_type=jnp.float32)

    @pl.when(k == 0)
    def _():
        acc_ref[...] = part

    @pl.when(k != 0)
    def _():
        acc_ref[...] = acc_ref[...] + part

    @pl.when(k == nk - 1)
    def _():
        h = jax.nn.gelu(acc_ref[...] + b1_ref[...])
        plog = jnp.dot(h, w2p_ref[...], preferred_element_type=jnp.float32)

        @pl.when(j == 0)
        def _():
            log_ref[...] = plog + b2p_ref[...]

        @pl.when(j != 0)
        def _():
            log_ref[...] = log_ref[...] + plog

        @pl.when(j == nj - 1)
        def _():
            lg = log_ref[...]
            m = jnp.max(lg)
            rows = jax.lax.broadcasted_iota(jnp.int32, lg.shape, 0)
            cols = jax.lax.broadcasted_iota(jnp.int32, lg.shape, 1)
            flat = rows * E + cols
            idx = jnp.min(jnp.where(lg == m, flat, jnp.int32(2**30)))
            out_ref[...] = tab_ref[0, pl.ds(idx, 1), :]


def kernel(predicate, W1, b1, W2, b2, expert_tables, input):
    T, D = predicate.shape
    H = W1.shape[1]
    E = W2.shape[1]
    n_tab, ROWS, ED = expert_tables.shape

    HB = 1024
    DB = 1024
    J = H // HB
    K = D // DB

    W2p = jnp.zeros((H, _EPAD), jnp.float32).at[:, :E].set(W2)
    b2p = jnp.full((1, _EPAD), -1e30, jnp.float32).at[0, :E].set(b2)
    b1r = b1.reshape(1, H)
    sp = jnp.asarray(input, jnp.int32).reshape(1)

    grid_spec = pltpu.PrefetchScalarGridSpec(
        num_scalar_prefetch=1,
        grid=(J, K),
        in_specs=[
            pl.BlockSpec((T, DB), lambda j, k, sp: (0, k)),
            pl.BlockSpec((DB, HB), lambda j, k, sp: (k, j)),
            pl.BlockSpec((1, HB), lambda j, k, sp: (0, j)),
            pl.BlockSpec((HB, _EPAD), lambda j, k, sp: (j, 0)),
            pl.BlockSpec((1, _EPAD), lambda j, k, sp: (0, 0)),
            pl.BlockSpec((1, ROWS, ED), lambda j, k, sp: (sp[0], 0, 0)),
        ],
        out_specs=pl.BlockSpec((1, ED), lambda j, k, sp: (0, 0)),
        scratch_shapes=[
            pltpu.VMEM((T, HB), jnp.float32),
            pltpu.VMEM((T, _EPAD), jnp.float32),
        ],
    )

    out = pl.pallas_call(
        functools.partial(_body, E),
        grid_spec=grid_spec,
        out_shape=jax.ShapeDtypeStruct((1, ED), jnp.float32),
        compiler_params=pltpu.CompilerParams(
            dimension_semantics=("arbitrary", "arbitrary"),
        ),
    )(sp, predicate, W1, b1r, W2p, b2p, expert_tables)
    return out.reshape(ED)


# speedup vs baseline: 1.0042x; 1.0042x over previous
import functools

import jax
import jax.numpy as jnp
from jax.experimental import pallas as pl
from jax.experimental.pallas import tpu as pltpu

_EPAD = 128


def _body(E, sp_ref, x_ref, w1_ref, b1_ref, w2p_ref, b2p_ref, tab_ref,
          out_ref, acc_ref, log_ref):
    j = pl.program_id(0)
    k = pl.program_id(1)
    nj = pl.num_programs(0)
    nk = pl.num_programs(1)

    part = jnp.dot(x_ref[...].astype(jnp.bfloat16),
                   w1_ref[...].astype(jnp.bfloat16),
                   preferred_element_type=jnp.float32)

    @pl.when(k == 0)
    def _():
        acc_ref[...] = part

    @pl.when(k != 0)
    def _():
        acc_ref[...] = acc_ref[...] + part

    @pl.when(k == nk - 1)
    def _():
        h = jax.nn.gelu(acc_ref[...] + b1_ref[...])
        plog = jnp.dot(h.astype(jnp.bfloat16),
                       w2p_ref[...].astype(jnp.bfloat16),
                       preferred_element_type=jnp.float32)

        @pl.when(j == 0)
        def _():
            log_ref[...] = plog + b2p_ref[...]

        @pl.when(j != 0)
        def _():
            log_ref[...] = log_ref[...] + plog

        @pl.when(j == nj - 1)
        def _():
            lg = log_ref[...]
            m = jnp.max(lg)
            rows = jax.lax.broadcasted_iota(jnp.int32, lg.shape, 0)
            cols = jax.lax.broadcasted_iota(jnp.int32, lg.shape, 1)
            flat = rows * E + cols
            idx = jnp.min(jnp.where(lg == m, flat, jnp.int32(2**30)))
            out_ref[...] = tab_ref[0, pl.ds(idx, 1), :]


def kernel(predicate, W1, b1, W2, b2, expert_tables, input):
    T, D = predicate.shape
    H = W1.shape[1]
    E = W2.shape[1]
    n_tab, ROWS, ED = expert_tables.shape

    HB = 1024
    DB = 1024
    J = H // HB
    K = D // DB

    W2p = jnp.zeros((H, _EPAD), jnp.float32).at[:, :E].set(W2)
    b2p = jnp.full((1, _EPAD), -1e30, jnp.float32).at[0, :E].set(b2)
    b1r = b1.reshape(1, H)
    sp = jnp.asarray(input, jnp.int32).reshape(1)

    grid_spec = pltpu.PrefetchScalarGridSpec(
        num_scalar_prefetch=1,
        grid=(J, K),
        in_specs=[
            pl.BlockSpec((T, DB), lambda j, k, sp: (0, k)),
            pl.BlockSpec((DB, HB), lambda j, k, sp: (k, j)),
            pl.BlockSpec((1, HB), lambda j, k, sp: (0, j)),
            pl.BlockSpec((HB, _EPAD), lambda j, k, sp: (j, 0)),
            pl.BlockSpec((1, _EPAD), lambda j, k, sp: (0, 0)),
            pl.BlockSpec((1, ROWS, ED), lambda j, k, sp: (sp[0], 0, 0)),
        ],
        out_specs=pl.BlockSpec((1, ED), lambda j, k, sp: (0, 0)),
        scratch_shapes=[
            pltpu.VMEM((T, HB), jnp.float32),
            pltpu.VMEM((T, _EPAD), jnp.float32),
        ],
    )

    out = pl.pallas_call(
        functools.partial(_body, E),
        grid_spec=grid_spec,
        out_shape=jax.ShapeDtypeStruct((1, ED), jnp.float32),
        compiler_params=pltpu.CompilerParams(
            dimension_semantics=("arbitrary", "arbitrary"),
        ),
    )(sp, predicate, W1, b1r, W2p, b2p, expert_tables)
    return out.reshape(ED)
